# baseline (speedup 1.0000x reference)
import functools

import jax
import jax.numpy as jnp
from jax import lax
from jax.experimental import pallas as pl
from jax.experimental.pallas import tpu as pltpu
from jax.experimental.pallas import tpu_sc as plsc

NC = 2
NS = 16
NW = NC * NS
CH = 128
G = 8


def _make_scatter_bf16(acc_rows, d, nch):
    mesh = plsc.VectorSubcoreMesh(core_axis_name="c", subcore_axis_name="s")
    rpt = acc_rows // NS
    dw = d // 2
    nb = d // 32

    @functools.partial(
        pl.kernel,
        out_type=jax.ShapeDtypeStruct((NC, acc_rows, d), jnp.float32),
        mesh=mesh,
        scratch_types=[
            pltpu.VMEM((G, CH), jnp.int32),
            pltpu.VMEM((G, CH), jnp.int32),
            pltpu.VMEM((CH, dw), jnp.int32),
            pltpu.VMEM((CH, dw), jnp.int32),
            pltpu.VMEM((CH, d), jnp.float32),
            pltpu.VMEM_SHARED((acc_rows, d), jnp.float32),
            pltpu.SemaphoreType.DMA,
            pltpu.SemaphoreType.DMA,
        ],
        compiler_params=pltpu.CompilerParams(
            use_tc_tiling_on_sc=False, needs_layout_passes=False),
    )
    def scat(src_hbm, row_hbm, col_hbm, zero_hbm, out_hbm,
             row_v, col_v, pb0, pb1, fb, acc, gsem, ssem):
        cid = lax.axis_index("c")
        sid = lax.axis_index("s")
        cbase = (sid * NC + cid) * nch
        pltpu.sync_copy(zero_hbm.at[pl.ds(sid * rpt, rpt)],
                        acc.at[pl.ds(sid * rpt, rpt)])
        plsc.subcore_barrier()
        pbufs = (pb0, pb1)

        def convert(pb):
            def crow(i, carry):
                for k in range(nb):
                    w16 = pb[i, pl.ds(k * 16, 16)]
                    ab = plsc.bitcast(w16, jnp.bfloat16)
                    a, b = plsc.unpack(ab, format=plsc.PackFormat.INTERLEAVED)
                    fb[i, pl.ds(k * 16, 16)] = a
                    fb[i, pl.ds(dw + k * 16, 16)] = b
                return carry

            lax.fori_loop(0, CH, crow, 0)

        def group(g, carry):
            base = pl.multiple_of(cbase + g * G, G)
            pltpu.sync_copy(row_hbm.at[pl.ds(base, G)], row_v)
            pltpu.sync_copy(col_hbm.at[pl.ds(base, G)], col_v)
            gd = pltpu.async_copy(src_hbm.at[row_v.at[0]], pbufs[0], gsem)
            sd_prev = None
            for jj in range(G):
                gd.wait()
                if jj + 1 < G:
                    gd = pltpu.async_copy(src_hbm.at[row_v.at[jj + 1]],
                                          pbufs[(jj + 1) % 2], gsem)
                if sd_prev is not None:
                    sd_prev.wait()
                convert(pbufs[jj % 2])
                sd_prev = pltpu.async_copy(fb, acc.at[col_v.at[jj]],
                                           ssem, add=True)
            sd_prev.wait()
            return carry

        lax.fori_loop(0, nch // G, group, 0)

        plsc.subcore_barrier()
        pltpu.sync_copy(acc.at[pl.ds(sid * rpt, rpt)],
                        out_hbm.at[cid].at[pl.ds(sid * rpt, rpt)])

    return scat


def _make_deghist(n_hist, nch):
    mesh = plsc.VectorSubcoreMesh(core_axis_name="c", subcore_axis_name="s")

    @functools.partial(
        pl.kernel,
        out_type=jax.ShapeDtypeStruct((NW, n_hist), jnp.float32),
        mesh=mesh,
        scratch_types=[
            pltpu.VMEM((nch, CH), jnp.int32),
            pltpu.VMEM((n_hist,), jnp.float32),
        ],
        compiler_params=pltpu.CompilerParams(needs_layout_passes=False),
    )
    def deg(col_hbm, out_hbm, col_v, hist):
        cid = lax.axis_index("c")
        sid = lax.axis_index("s")
        wid = sid * NC + cid
        pltpu.sync_copy(col_hbm.at[pl.ds(wid * nch, nch)], col_v)

        def zbody(i, carry):
            hist[pl.ds(i * 16, 16)] = jnp.zeros((16,), jnp.float32)
            return carry

        lax.fori_loop(0, n_hist // 16, zbody, 0)

        ones = jnp.ones((16,), jnp.float32)

        def ebody(j, carry):
            for k in range(CH // 16):
                idx = col_v[j, pl.ds(k * 16, 16)]
                plsc.addupdate_scatter(hist, [idx], ones)
            return carry

        lax.fori_loop(0, nch, ebody, 0)

        pltpu.sync_copy(hist, out_hbm.at[wid])

    return deg


def _deg_of(d_ref):
    return jnp.sum(d_ref[...], axis=1, keepdims=True) + 1.0


def _prep_body(x_ref, w0_ref, wlo_ref, whi_ref, d_ref,
               h0_ref, ulo_ref, uhi_ref):
    dis = lax.rsqrt(_deg_of(d_ref))
    x = x_ref[...]
    h0_ref[...] = jnp.dot(x, w0_ref[...], preferred_element_type=jnp.float32)
    ulo_ref[...] = jnp.dot(x, wlo_ref[...],
                           preferred_element_type=jnp.float32) * dis
    uhi_ref[...] = jnp.dot(x, whi_ref[...],
                           preferred_element_type=jnp.float32) * dis


def _mid_body(z0_ref, z1_ref, ulo_ref, uhi_ref, d_ref,
              h1_ref, tlo_ref, thi_ref, hop):
    deg = _deg_of(d_ref)
    dis = lax.rsqrt(deg)
    hh = hop // 2
    zf_lo = z0_ref[:, :hop] + z1_ref[:, :hop] + ulo_ref[...]
    zf_hi = z0_ref[:, hop:] + z1_ref[:, hop:] + uhi_ref[...]
    h1_ref[...] = jnp.concatenate(
        [zf_lo[:, 0:16], zf_hi[:, 0:16], zf_lo[:, 16:32], zf_hi[:, 16:32]],
        axis=1) * dis
    tlo_ref[...] = zf_lo[:, hh:hop] / deg
    thi_ref[...] = zf_hi[:, hh:hop] / deg


def _final_body(w0_ref, w1_ref, tlo_ref, thi_ref, h0_ref, h1_ref, d_ref,
                b0_ref, b1_ref, b2_ref, wout_ref, bout_ref, o_ref, hop):
    dis = lax.rsqrt(_deg_of(d_ref))
    hh = hop // 2
    wf_lo = w0_ref[:, :hh] + w1_ref[:, :hh] + tlo_ref[...]
    wf_hi = w0_ref[:, hh:] + w1_ref[:, hh:] + thi_ref[...]
    h2 = jnp.concatenate(
        [wf_lo[:, 0:16], wf_hi[:, 0:16], wf_lo[:, 16:32], wf_hi[:, 16:32]],
        axis=1) * dis
    h = jnp.concatenate([h0_ref[...] + b0_ref[...],
                         h1_ref[...] + b1_ref[...],
                         h2 + b2_ref[...]], axis=1)
    h = jnp.maximum(h, 0.0)
    o_ref[...] = jnp.dot(h, wout_ref[...], preferred_element_type=jnp.float32) \
        + bout_ref[...]


def _pack_bf16(lo, hi):
    st = jnp.stack([lo.astype(jnp.bfloat16), hi.astype(jnp.bfloat16)],
                   axis=-1)
    return lax.bitcast_convert_type(st, jnp.int32)


def kernel(x, edge_index, W0, b0, W1, b1, W2, b2, Wout, bout):
    n, in_ch = x.shape
    hop = W0.shape[1]
    out_ch = Wout.shape[1]
    e = edge_index.shape[1]

    per_w = -(-e // (NW * CH * G)) * CH * G
    nch = per_w // CH
    pad = per_w * NW - e
    acc_rows = -(-(n + 1) // (NS * 8)) * (NS * 8)

    row = jnp.concatenate([edge_index[0], jnp.zeros((pad,), edge_index.dtype)])
    col = jnp.concatenate([edge_index[1], jnp.full((pad,), n, edge_index.dtype)])
    row_p = row.reshape(NW * nch, CH)
    col_p = col.reshape(NW * nch, CH)

    zeros2h = jnp.zeros((acc_rows, 2 * hop), jnp.float32)
    zerosh = jnp.zeros((acc_rows, hop), jnp.float32)

    hists = _make_deghist(acc_rows, nch)(col_p)
    dT = hists[:, :n].T

    wcat = jnp.concatenate([W1, W2], axis=1)
    qs = [q * 32 for q in range(2 * hop // 32)]
    lo_idx = jnp.array([q + r for q in qs for r in range(16)], jnp.int32)
    wlo = wcat[:, lo_idx]
    whi = wcat[:, lo_idx + 16]

    R = 2000
    grid = (n // R,)
    blk = lambda r, c: pl.BlockSpec((r, c), lambda i: (i, 0))
    full = lambda r, c: pl.BlockSpec((r, c), lambda i: (0, 0))
    h0, ulo, uhi = pl.pallas_call(
        _prep_body,
        grid=grid,
        in_specs=[blk(R, in_ch), full(in_ch, hop), full(in_ch, hop),
                  full(in_ch, hop), blk(R, NW)],
        out_specs=[blk(R, hop), blk(R, hop), blk(R, hop)],
        out_shape=[jax.ShapeDtypeStruct((n, hop), jnp.float32)] * 3,
    )(x, W0, wlo, whi, dT)

    upk = _pack_bf16(ulo, uhi)
    zp = _make_scatter_bf16(acc_rows, 2 * hop, nch)(upk, row_p, col_p, zeros2h)

    h1, tlo, thi = pl.pallas_call(
        functools.partial(_mid_body, hop=hop),
        grid=grid,
        in_specs=[blk(R, 2 * hop), blk(R, 2 * hop), blk(R, hop),
                  blk(R, hop), blk(R, NW)],
        out_specs=[blk(R, hop), blk(R, hop // 2), blk(R, hop // 2)],
        out_shape=[jax.ShapeDtypeStruct((n, hop), jnp.float32),
                   jax.ShapeDtypeStruct((n, hop // 2), jnp.float32),
                   jax.ShapeDtypeStruct((n, hop // 2), jnp.float32)],
    )(zp[0, :n, :], zp[1, :n, :], ulo, uhi, dT)

    tpk = _pack_bf16(tlo, thi)
    wp = _make_scatter_bf16(acc_rows, hop, nch)(tpk, row_p, col_p, zerosh)

    out = pl.pallas_call(
        functools.partial(_final_body, hop=hop),
        grid=grid,
        in_specs=[blk(R, hop), blk(R, hop), blk(R, hop // 2), blk(R, hop // 2),
                  blk(R, hop), blk(R, hop), blk(R, NW),
                  full(1, hop), full(1, hop), full(1, hop),
                  full(3 * hop, out_ch), full(1, out_ch)],
        out_specs=blk(R, out_ch),
        out_shape=jax.ShapeDtypeStruct((n, out_ch), jnp.float32),
    )(wp[0, :n, :], wp[1, :n, :], tlo, thi, h0, h1, dT,
      b0.reshape(1, hop), b1.reshape(1, hop), b2.reshape(1, hop),
      Wout, bout.reshape(1, out_ch))
    return out


# speedup vs baseline: 1.0888x; 1.0888x over previous
import functools

import jax
import jax.numpy as jnp
from jax import lax
from jax.experimental import pallas as pl
from jax.experimental.pallas import tpu as pltpu
from jax.experimental.pallas import tpu_sc as plsc

NC = 2
NS = 16
NW = NC * NS
CH = 128
G = 8


def _make_scatter_bf16(acc_rows, d, nch_a, nch_b):
    mesh = plsc.VectorSubcoreMesh(core_axis_name="c", subcore_axis_name="s")
    rpt = acc_rows // NS
    dw = d // 2
    nb = d // 32

    @functools.partial(
        pl.kernel,
        out_type=jax.ShapeDtypeStruct((NC, acc_rows, d), jnp.float32),
        mesh=mesh,
        scratch_types=[
            pltpu.VMEM((G, CH), jnp.int32),
            pltpu.VMEM((G, CH), jnp.int32),
            pltpu.VMEM((CH, dw), jnp.int32),
            pltpu.VMEM((CH, dw), jnp.int32),
            pltpu.VMEM((CH, d), jnp.float32),
            pltpu.VMEM_SHARED((acc_rows, d), jnp.float32),
            pltpu.SemaphoreType.DMA,
            pltpu.SemaphoreType.DMA,
        ],
        compiler_params=pltpu.CompilerParams(
            use_tc_tiling_on_sc=False, needs_layout_passes=False),
    )
    def scat(src_hbm, row_a, col_a, row_b, col_b, zero_hbm, out_hbm,
             row_v, col_v, pb0, pb1, fb, acc, gsem, ssem):
        cid = lax.axis_index("c")
        sid = lax.axis_index("s")
        pltpu.sync_copy(zero_hbm.at[pl.ds(sid * rpt, rpt)],
                        acc.at[pl.ds(sid * rpt, rpt)])
        plsc.subcore_barrier()
        pbufs = (pb0, pb1)

        def convert(pb):
            def crow(i, carry):
                for k in range(nb):
                    w16 = pb[i, pl.ds(k * 16, 16)]
                    ab = plsc.bitcast(w16, jnp.bfloat16)
                    a, b = plsc.unpack(ab, format=plsc.PackFormat.INTERLEAVED)
                    fb[i, pl.ds(k * 16, 16)] = a
                    fb[i, pl.ds(dw + k * 16, 16)] = b
                return carry

            lax.fori_loop(0, CH, crow, 0)

        def make_group(row_hbm, col_hbm):
            def group(g, carry):
                base = pl.multiple_of(g * G, G)
                pltpu.sync_copy(row_hbm.at[sid].at[pl.ds(base, G)], row_v)
                pltpu.sync_copy(col_hbm.at[sid].at[pl.ds(base, G)], col_v)
                gd = pltpu.async_copy(src_hbm.at[row_v.at[0]], pbufs[0], gsem)
                sd_prev = None
                for jj in range(G):
                    gd.wait()
                    if jj + 1 < G:
                        gd = pltpu.async_copy(src_hbm.at[row_v.at[jj + 1]],
                                              pbufs[(jj + 1) % 2], gsem)
                    if sd_prev is not None:
                        sd_prev.wait()
                    convert(pbufs[jj % 2])
                    sd_prev = pltpu.async_copy(fb, acc.at[col_v.at[jj]],
                                               ssem, add=True)
                sd_prev.wait()
                return carry

            return group

        @pl.when(cid == 0)
        def _():
            lax.fori_loop(0, nch_a // G, make_group(row_a, col_a), 0)

        @pl.when(cid == 1)
        def _():
            lax.fori_loop(0, nch_b // G, make_group(row_b, col_b), 0)

        plsc.subcore_barrier()
        pltpu.sync_copy(acc.at[pl.ds(sid * rpt, rpt)],
                        out_hbm.at[cid].at[pl.ds(sid * rpt, rpt)])

    return scat


def _make_deghist(n_hist, nch):
    mesh = plsc.VectorSubcoreMesh(core_axis_name="c", subcore_axis_name="s")

    @functools.partial(
        pl.kernel,
        out_type=jax.ShapeDtypeStruct((NW, n_hist), jnp.float32),
        mesh=mesh,
        scratch_types=[
            pltpu.VMEM((nch, CH), jnp.int32),
            pltpu.VMEM((n_hist,), jnp.float32),
        ],
        compiler_params=pltpu.CompilerParams(needs_layout_passes=False),
    )
    def deg(col_hbm, out_hbm, col_v, hist):
        cid = lax.axis_index("c")
        sid = lax.axis_index("s")
        wid = sid * NC + cid
        pltpu.sync_copy(col_hbm.at[pl.ds(wid * nch, nch)], col_v)

        def zbody(i, carry):
            hist[pl.ds(i * 16, 16)] = jnp.zeros((16,), jnp.float32)
            return carry

        lax.fori_loop(0, n_hist // 16, zbody, 0)

        ones = jnp.ones((16,), jnp.float32)

        def ebody(j, carry):
            for k in range(CH // 16):
                idx = col_v[j, pl.ds(k * 16, 16)]
                plsc.addupdate_scatter(hist, [idx], ones)
            return carry

        lax.fori_loop(0, nch, ebody, 0)

        pltpu.sync_copy(hist, out_hbm.at[wid])

    return deg


def _deg_of(d_ref):
    return jnp.sum(d_ref[...], axis=1, keepdims=True) + 1.0


def _prep_body(x_ref, w0_ref, wlo_ref, whi_ref, d_ref,
               h0_ref, ulo_ref, uhi_ref):
    dis = lax.rsqrt(_deg_of(d_ref))
    x = x_ref[...]
    h0_ref[...] = jnp.dot(x, w0_ref[...], preferred_element_type=jnp.float32)
    ulo_ref[...] = jnp.dot(x, wlo_ref[...],
                           preferred_element_type=jnp.float32) * dis
    uhi_ref[...] = jnp.dot(x, whi_ref[...],
                           preferred_element_type=jnp.float32) * dis


def _mid_body(z0_ref, z1_ref, ulo_ref, uhi_ref, d_ref,
              h1_ref, tlo_ref, thi_ref, hop):
    deg = _deg_of(d_ref)
    dis = lax.rsqrt(deg)
    hh = hop // 2
    zf_lo = z0_ref[:, :hop] + z1_ref[:, :hop] + ulo_ref[...]
    zf_hi = z0_ref[:, hop:] + z1_ref[:, hop:] + uhi_ref[...]
    h1_ref[...] = jnp.concatenate(
        [zf_lo[:, 0:16], zf_hi[:, 0:16], zf_lo[:, 16:32], zf_hi[:, 16:32]],
        axis=1) * dis
    tlo_ref[...] = zf_lo[:, hh:hop] / deg
    thi_ref[...] = zf_hi[:, hh:hop] / deg


def _final_body(w0_ref, w1_ref, tlo_ref, thi_ref, h0_ref, h1_ref, d_ref,
                b0_ref, b1_ref, b2_ref, wout_ref, bout_ref, o_ref, hop):
    dis = lax.rsqrt(_deg_of(d_ref))
    hh = hop // 2
    wf_lo = w0_ref[:, :hh] + w1_ref[:, :hh] + tlo_ref[...]
    wf_hi = w0_ref[:, hh:] + w1_ref[:, hh:] + thi_ref[...]
    h2 = jnp.concatenate(
        [wf_lo[:, 0:16], wf_hi[:, 0:16], wf_lo[:, 16:32], wf_hi[:, 16:32]],
        axis=1) * dis
    h = jnp.concatenate([h0_ref[...] + b0_ref[...],
                         h1_ref[...] + b1_ref[...],
                         h2 + b2_ref[...]], axis=1)
    h = jnp.maximum(h, 0.0)
    o_ref[...] = jnp.dot(h, wout_ref[...], preferred_element_type=jnp.float32) \
        + bout_ref[...]


def _pack_bf16(lo, hi):
    st = jnp.stack([lo.astype(jnp.bfloat16), hi.astype(jnp.bfloat16)],
                   axis=-1)
    return lax.bitcast_convert_type(st, jnp.int32)


def kernel(x, edge_index, W0, b0, W1, b1, W2, b2, Wout, bout):
    n, in_ch = x.shape
    hop = W0.shape[1]
    out_ch = Wout.shape[1]
    e = edge_index.shape[1]

    per_w = -(-e // (NW * CH * G)) * CH * G
    nch = per_w // CH
    pad = per_w * NW - e
    nch_a = (2 * nch * 55 // 100) // G * G
    nch_b = 2 * nch - nch_a
    ea = NS * nch_a * CH
    acc_rows = -(-(n + 1) // (NS * 8)) * (NS * 8)

    row = jnp.concatenate([edge_index[0], jnp.zeros((pad,), edge_index.dtype)])
    col = jnp.concatenate([edge_index[1], jnp.full((pad,), n, edge_index.dtype)])
    row_a = row[:ea].reshape(NS, nch_a, CH)
    col_a = col[:ea].reshape(NS, nch_a, CH)
    row_b = row[ea:].reshape(NS, nch_b, CH)
    col_b = col[ea:].reshape(NS, nch_b, CH)
    col_p = col.reshape(NW * nch, CH)

    zeros2h = jnp.zeros((acc_rows, 2 * hop), jnp.float32)
    zerosh = jnp.zeros((acc_rows, hop), jnp.float32)

    hists = _make_deghist(acc_rows, nch)(col_p)
    dT = hists[:, :n].T

    wcat = jnp.concatenate([W1, W2], axis=1)
    qs = [q * 32 for q in range(2 * hop // 32)]
    lo_idx = jnp.array([q + r for q in qs for r in range(16)], jnp.int32)
    wlo = wcat[:, lo_idx]
    whi = wcat[:, lo_idx + 16]

    R = 2000
    grid = (n // R,)
    blk = lambda r, c: pl.BlockSpec((r, c), lambda i: (i, 0))
    full = lambda r, c: pl.BlockSpec((r, c), lambda i: (0, 0))
    h0, ulo, uhi = pl.pallas_call(
        _prep_body,
        grid=grid,
        in_specs=[blk(R, in_ch), full(in_ch, hop), full(in_ch, hop),
                  full(in_ch, hop), blk(R, NW)],
        out_specs=[blk(R, hop), blk(R, hop), blk(R, hop)],
        out_shape=[jax.ShapeDtypeStruct((n, hop), jnp.float32)] * 3,
    )(x, W0, wlo, whi, dT)

    upk = _pack_bf16(ulo, uhi)
    zp = _make_scatter_bf16(acc_rows, 2 * hop, nch_a, nch_b)(
        upk, row_a, col_a, row_b, col_b, zeros2h)

    h1, tlo, thi = pl.pallas_call(
        functools.partial(_mid_body, hop=hop),
        grid=grid,
        in_specs=[blk(R, 2 * hop), blk(R, 2 * hop), blk(R, hop),
                  blk(R, hop), blk(R, NW)],
        out_specs=[blk(R, hop), blk(R, hop // 2), blk(R, hop // 2)],
        out_shape=[jax.ShapeDtypeStruct((n, hop), jnp.float32),
                   jax.ShapeDtypeStruct((n, hop // 2), jnp.float32),
                   jax.ShapeDtypeStruct((n, hop // 2), jnp.float32)],
    )(zp[0, :n, :], zp[1, :n, :], ulo, uhi, dT)

    tpk = _pack_bf16(tlo, thi)
    wp = _make_scatter_bf16(acc_rows, hop, nch_a, nch_b)(
        tpk, row_a, col_a, row_b, col_b, zerosh)

    out = pl.pallas_call(
        functools.partial(_final_body, hop=hop),
        grid=grid,
        in_specs=[blk(R, hop), blk(R, hop), blk(R, hop // 2), blk(R, hop // 2),
                  blk(R, hop), blk(R, hop), blk(R, NW),
                  full(1, hop), full(1, hop), full(1, hop),
                  full(3 * hop, out_ch), full(1, out_ch)],
        out_specs=blk(R, out_ch),
        out_shape=jax.ShapeDtypeStruct((n, out_ch), jnp.float32),
    )(wp[0, :n, :], wp[1, :n, :], tlo, thi, h0, h1, dT,
      b0.reshape(1, hop), b1.reshape(1, hop), b2.reshape(1, hop),
      Wout, bout.reshape(1, out_ch))
    return out
